# Initial kernel scaffold; baseline (speedup 1.0000x reference)
import jax
import jax.numpy as jnp
from jax.experimental import pallas as pl


def kernel(x, edge_index, W1, a_src1, a_dst1, b1, W2, a_src2, a_dst2, b2):
    raise NotImplementedError("write your pallas kernel here")


# speedup vs baseline: 1.0297x; 1.0297x over previous
import functools

import jax
import jax.numpy as jnp
from jax.experimental import pallas as pl
from jax.experimental.pallas import tpu as pltpu

N = 10000
E = 320000
D_IN = 128
HID = 128
H1 = 8
NUM_CLASSES = 40
NEG_SLOPE = 0.2


def _matmul_kernel(x_ref, w_ref, o_ref):
    o_ref[...] = jnp.dot(x_ref[...], w_ref[...],
                         preferred_element_type=jnp.float32)


def _pallas_matmul(x, w, block_m=1000):
    m, k = x.shape
    k2, n = w.shape
    grid = (m // block_m,)
    return pl.pallas_call(
        _matmul_kernel,
        grid=grid,
        in_specs=[
            pl.BlockSpec((block_m, k), lambda i: (i, 0)),
            pl.BlockSpec((k, n), lambda i: (0, 0)),
        ],
        out_specs=pl.BlockSpec((block_m, n), lambda i: (i, 0)),
        out_shape=jax.ShapeDtypeStruct((m, n), jnp.float32),
    )(x, w)


def _gat_layer(h2d, src, dst, att_src, att_dst, heads, out_ch):
    n = h2d.shape[0]
    h = h2d.reshape(n, heads, out_ch)
    a_s = (h * att_src[None]).sum(-1)
    a_d = (h * att_dst[None]).sum(-1)
    alpha = a_s[src] + a_d[dst]
    alpha = jnp.where(alpha > 0, alpha, NEG_SLOPE * alpha)
    amax = jax.ops.segment_max(alpha, dst, num_segments=n)
    amax = jnp.where(jnp.isfinite(amax), amax, 0.0)
    ex = jnp.exp(alpha - amax[dst])
    denom = jax.ops.segment_sum(ex, dst, num_segments=n)
    coef = ex / (denom[dst] + 1e-16)
    msg = h[src] * coef[:, :, None]
    out = jax.ops.segment_sum(msg, dst, num_segments=n)
    return out


def kernel(x, edge_index, W1, a_src1, a_dst1, b1, W2, a_src2, a_dst2, b2):
    loop = jnp.arange(N, dtype=edge_index.dtype)
    src = jnp.concatenate([edge_index[0], loop])
    dst = jnp.concatenate([edge_index[1], loop])

    h1 = _pallas_matmul(x, W1)
    out1 = _gat_layer(h1, src, dst, a_src1, a_dst1, H1, HID)
    out1 = out1.reshape(N, H1 * HID) + b1
    h2 = jax.nn.elu(out1)

    g = _pallas_matmul(h2, W2)
    out2 = _gat_layer(g, src, dst, a_src2, a_dst2, 1, NUM_CLASSES)
    out2 = out2.reshape(N, NUM_CLASSES) + b2
    return out2


# speedup vs baseline: 14.6761x; 14.2528x over previous
import jax
import jax.numpy as jnp
from jax import lax
from jax.experimental import pallas as pl
from jax.experimental.pallas import tpu as pltpu
from jax.experimental.pallas import tpu_sc as plsc

N = 10000
E = 320000
D_IN = 128
HID = 128
H1 = 8
NUM_CLASSES = 40
NEG_SLOPE = 0.2

N2 = 10240
NACC = 10112
EREAL = E + N
EBLK = 64
NROWS = 5184
EP = NROWS * EBLK
NTILES = 16
NCORES = 2
NB1 = NROWS // NTILES
NB2 = NROWS // (NTILES * NCORES)
ET1 = NB1 * EBLK
ET2 = NB2 * EBLK
BPR = 54
EPR = BPR * EBLK
RND1 = NB1 // BPR
RND2 = NB2 // BPR
RPT1 = NACC // NTILES
NP2 = N2 // 2
RPT2 = NP2 // NTILES
DND = 80
W1OUT = H1 * HID
HPC = H1 // NCORES


def _tc_a_body(x_ref, w1_ref, asm_ref, adm_ref, h_ref, ast_ref, adt_ref):
    h = jnp.dot(x_ref[...], w1_ref[...], preferred_element_type=jnp.float32)
    for hh in range(H1):
        h_ref[hh] = h[:, hh * HID:(hh + 1) * HID]
    dn = (((1,), (1,)), ((), ()))
    ast_ref[...] = lax.dot_general(asm_ref[...], h, dn,
                                   preferred_element_type=jnp.float32)
    adt_ref[...] = lax.dot_general(adm_ref[...], h, dn,
                                   preferred_element_type=jnp.float32)


def _tc_a(x_p, W1, asm, adm):
    bn = 1024
    grid = (N2 // bn,)
    return pl.pallas_call(
        _tc_a_body,
        grid=grid,
        in_specs=[
            pl.BlockSpec((bn, D_IN), lambda i: (i, 0)),
            pl.BlockSpec((D_IN, W1OUT), lambda i: (0, 0)),
            pl.BlockSpec((H1, W1OUT), lambda i: (0, 0)),
            pl.BlockSpec((H1, W1OUT), lambda i: (0, 0)),
        ],
        out_specs=[
            pl.BlockSpec((H1, bn, HID), lambda i: (0, i, 0)),
            pl.BlockSpec((H1, bn), lambda i: (0, i)),
            pl.BlockSpec((H1, bn), lambda i: (0, i)),
        ],
        out_shape=[
            jax.ShapeDtypeStruct((H1, N2, HID), jnp.float32),
            jax.ShapeDtypeStruct((H1, N2), jnp.float32),
            jax.ShapeDtypeStruct((H1, N2), jnp.float32),
        ],
    )(x_p, W1, asm, adm)


def _zero_rows(rb, nrows):
    zeros16 = jnp.zeros((16,), jnp.float32)

    def _zb(j, c):
        for ch in range(8):
            rb[j, pl.ds(ch * 16, 16)] = zeros16
        return c
    lax.fori_loop(0, nrows, _zb, 0)


def _zero_accum_slice(rb, accum, base, nrows):
    full, rem = nrows // EBLK, nrows % EBLK
    for k in range(full):
        pltpu.sync_copy(rb, accum.at[pl.ds(base + k * EBLK, EBLK)])
    if rem:
        pltpu.sync_copy(rb.at[pl.ds(0, rem)],
                        accum.at[pl.ds(base + full * EBLK, rem)])


def _sc_c1_body(src_hbm, dst_hbm, ast_hbm, adt_hbm, hflat_hbm,
                out_hbm, den_hbm,
                asb, adb, srcb, dstb, srcb2, dstb2, exb, rb,
                dnb, ridx, accum, dnshared):
    cid = lax.axis_index("c")
    tid = lax.axis_index("s")
    zeros16 = jnp.zeros((16,), jnp.float32)
    lane = lax.iota(jnp.int32, 16)

    def _zi(j, c):
        ridx[pl.ds(j * 16, 16)] = j * 16 + lane
        return c
    lax.fori_loop(0, DND // 16, _zi, 0)

    for hl in range(HPC):
        hh = cid * HPC + hl
        pltpu.sync_copy(ast_hbm.at[hh], asb)
        pltpu.sync_copy(adt_hbm.at[hh], adb)

        _zero_rows(rb, EBLK)

        def _zd(j, c):
            for ch in range(8):
                dnb[j, pl.ds(ch * 16, 16)] = zeros16
            return c
        lax.fori_loop(0, DND, _zd, 0)
        _zero_accum_slice(rb, accum, tid * RPT1, RPT1)

        @pl.when(tid == 0)
        def _():
            _zero_accum_slice(rb, dnshared, 0, DND)
        plsc.subcore_barrier()

        def _round(r, c0):
            e0 = tid * ET1 + r * EPR
            pltpu.sync_copy(src_hbm.at[pl.ds(e0, EPR)], srcb)
            pltpu.sync_copy(dst_hbm.at[pl.ds(e0, EPR)], dstb)

            def _blk(b, c):
                for g in range(4):
                    sv = srcb[pl.ds(b * EBLK + g * 16, 16)]
                    dv = dstb[pl.ds(b * EBLK + g * 16, 16)]
                    av = (plsc.load_gather(asb, [sv])
                          + plsc.load_gather(adb, [dv]))
                    av = jnp.where(av > 0, av, NEG_SLOPE * av)
                    exv = jnp.exp(av)
                    gid = e0 + b * EBLK + g * 16 + lane
                    exv = jnp.where(gid < EREAL, exv, 0.0)
                    exb[pl.ds(g * 16, 16)] = exv
                    srcb2[pl.ds(g * 16, 16)] = sv + hh * N2
                    dstb2[pl.ds(g * 16, 16)] = dv
                    dr = lax.shift_right_logical(dv, 7)
                    dc = jnp.bitwise_and(dv, 127)
                    for l in range(16):
                        plsc.addupdate_scatter(dnb, [dr, dc], exv,
                                               mask=lane == l)
                pltpu.sync_copy(hflat_hbm.at[srcb2], rb)

                def _srow(j, c2):
                    eb = plsc.load_gather(exb,
                                          [jnp.full((16,), j, jnp.int32)])
                    for ch in range(8):
                        rb[j, pl.ds(ch * 16, 16)] = (
                            rb[j, pl.ds(ch * 16, 16)] * eb)
                    return c2
                lax.fori_loop(0, EBLK, _srow, 0)
                pltpu.sync_copy(rb, accum.at[dstb2], add=True)
                return c
            lax.fori_loop(0, BPR, _blk, 0)
            return c0
        lax.fori_loop(0, RND1, _round, 0)

        pltpu.sync_copy(dnb, dnshared.at[ridx], add=True)
        plsc.subcore_barrier()

        @pl.when(tid == 0)
        def _():
            pltpu.sync_copy(dnshared, den_hbm.at[hh])

        pltpu.sync_copy(accum.at[pl.ds(tid * RPT1, RPT1)],
                        out_hbm.at[hh, pl.ds(tid * RPT1, RPT1)])
        plsc.subcore_barrier()


def _sc_c1(src1d, dst1d, ast, adt, hflat):
    mesh = plsc.VectorSubcoreMesh(core_axis_name="c", subcore_axis_name="s")
    f = pl.kernel(
        _sc_c1_body,
        out_type=[
            jax.ShapeDtypeStruct((H1, NACC, HID), jnp.float32),
            jax.ShapeDtypeStruct((H1, DND, 128), jnp.float32),
        ],
        mesh=mesh,
        compiler_params=pltpu.CompilerParams(needs_layout_passes=False),
        scratch_types=[
            pltpu.VMEM((N2,), jnp.float32),
            pltpu.VMEM((N2,), jnp.float32),
            pltpu.VMEM((EPR,), jnp.int32),
            pltpu.VMEM((EPR,), jnp.int32),
            pltpu.VMEM((EBLK,), jnp.int32),
            pltpu.VMEM((EBLK,), jnp.int32),
            pltpu.VMEM((EBLK,), jnp.float32),
            pltpu.VMEM((EBLK, HID), jnp.float32),
            pltpu.VMEM((DND, 128), jnp.float32),
            pltpu.VMEM((DND,), jnp.int32),
            pltpu.VMEM_SHARED((NACC, HID), jnp.float32),
            pltpu.VMEM_SHARED((DND, 128), jnp.float32),
        ],
    )
    return f(src1d, dst1d, ast, adt, hflat)


def _tc_d_body(s1_ref, dent_ref, b1_ref, w2_ref, a2s_ref, a2d_ref,
               g_ref, as2_ref, ad2_ref):
    cols = []
    for hh in range(H1):
        v = s1_ref[hh]
        den = dent_ref[:, hh:hh + 1]
        col = v * (1.0 / (den + 1e-16)) + b1_ref[hh, :]
        cols.append(jnp.where(col > 0, col,
                              jnp.exp(jnp.minimum(col, 0.0)) - 1.0))
    h2 = jnp.concatenate(cols, axis=1)
    g = jnp.dot(h2, w2_ref[...], preferred_element_type=jnp.float32)
    g_ref[...] = g
    as2_ref[...] = jnp.dot(g, a2s_ref[...], preferred_element_type=jnp.float32)
    ad2_ref[...] = jnp.dot(g, a2d_ref[...], preferred_element_type=jnp.float32)


def _tc_d(s1, dent, b1m, W2p, a2sp, a2dp):
    bn = 1024
    grid = (N2 // bn,)
    return pl.pallas_call(
        _tc_d_body,
        grid=grid,
        in_specs=[
            pl.BlockSpec((H1, bn, HID), lambda i: (0, i, 0)),
            pl.BlockSpec((bn, H1), lambda i: (i, 0)),
            pl.BlockSpec((H1, HID), lambda i: (0, 0)),
            pl.BlockSpec((W1OUT, HID), lambda i: (0, 0)),
            pl.BlockSpec((HID, 1), lambda i: (0, 0)),
            pl.BlockSpec((HID, 1), lambda i: (0, 0)),
        ],
        out_specs=[
            pl.BlockSpec((bn, HID), lambda i: (i, 0)),
            pl.BlockSpec((bn, 1), lambda i: (i, 0)),
            pl.BlockSpec((bn, 1), lambda i: (i, 0)),
        ],
        out_shape=[
            jax.ShapeDtypeStruct((N2, HID), jnp.float32),
            jax.ShapeDtypeStruct((N2, 1), jnp.float32),
            jax.ShapeDtypeStruct((N2, 1), jnp.float32),
        ],
    )(s1, dent, b1m, W2p, a2sp, a2dp)


def _sc_c2_body(src_hbm, dst_hbm, as2_hbm, ad2_hbm, g_hbm, out_hbm,
                asb, adb, srcb, dstb, srcb3, dstb2, prb, exb, rb, accum):
    cid = lax.axis_index("c")
    tid = lax.axis_index("s")
    wbase = (cid * NTILES + tid) * ET2
    pltpu.sync_copy(as2_hbm, asb)
    pltpu.sync_copy(ad2_hbm, adb)
    lane = lax.iota(jnp.int32, 16)

    _zero_rows(rb, EBLK)
    _zero_accum_slice(rb, accum, tid * RPT2, RPT2)
    plsc.subcore_barrier()

    def _round(r, c0):
        e0 = wbase + r * EPR
        pltpu.sync_copy(src_hbm.at[pl.ds(e0, EPR)], srcb)
        pltpu.sync_copy(dst_hbm.at[pl.ds(e0, EPR)], dstb)

        def _blk(b, c):
            for g in range(4):
                sv = srcb[pl.ds(b * EBLK + g * 16, 16)]
                dv = dstb[pl.ds(b * EBLK + g * 16, 16)]
                av = (plsc.load_gather(asb, [sv])
                      + plsc.load_gather(adb, [dv]))
                av = jnp.where(av > 0, av, NEG_SLOPE * av)
                exv = jnp.exp(av)
                gid = e0 + b * EBLK + g * 16 + lane
                exv = jnp.where(gid < EREAL, exv, 0.0)
                exb[pl.ds(g * 16, 16)] = exv
                srcb3[pl.ds(g * 16, 16)] = sv
                dstb2[pl.ds(g * 16, 16)] = lax.shift_right_logical(dv, 1)
                prb[pl.ds(g * 16, 16)] = jnp.bitwise_and(dv, 1).astype(
                    jnp.float32)
            pltpu.sync_copy(g_hbm.at[srcb3], rb)

            def _srow(j, c2):
                jj = jnp.full((16,), j, jnp.int32)
                eb = plsc.load_gather(exb, [jj])
                pb = plsc.load_gather(prb, [jj])
                odd = pb > 0.5
                for ch in range(4):
                    v = rb[j, pl.ds(ch * 16, 16)] * eb
                    if ch == 2:
                        v = v + jnp.where(lane == 8, eb, 0.0)
                    rb[j, pl.ds(ch * 16, 16)] = jnp.where(odd, 0.0, v)
                    rb[j, pl.ds((ch + 4) * 16, 16)] = jnp.where(odd, v, 0.0)
                return c2
            lax.fori_loop(0, EBLK, _srow, 0)
            pltpu.sync_copy(rb, accum.at[dstb2], add=True)
            return c
        lax.fori_loop(0, BPR, _blk, 0)
        return c0
    lax.fori_loop(0, RND2, _round, 0)
    plsc.subcore_barrier()

    pltpu.sync_copy(accum.at[pl.ds(tid * RPT2, RPT2)],
                    out_hbm.at[cid, pl.ds(tid * RPT2, RPT2)])


def _sc_c2(src1d, dst1d, as2, ad2, g):
    mesh = plsc.VectorSubcoreMesh(core_axis_name="c", subcore_axis_name="s")
    f = pl.kernel(
        _sc_c2_body,
        out_type=jax.ShapeDtypeStruct((NCORES, NP2, HID), jnp.float32),
        mesh=mesh,
        compiler_params=pltpu.CompilerParams(needs_layout_passes=False),
        scratch_types=[
            pltpu.VMEM((N2,), jnp.float32),
            pltpu.VMEM((N2,), jnp.float32),
            pltpu.VMEM((EPR,), jnp.int32),
            pltpu.VMEM((EPR,), jnp.int32),
            pltpu.VMEM((EBLK,), jnp.int32),
            pltpu.VMEM((EBLK,), jnp.int32),
            pltpu.VMEM((EBLK,), jnp.float32),
            pltpu.VMEM((EBLK,), jnp.float32),
            pltpu.VMEM((EBLK, HID), jnp.float32),
            pltpu.VMEM_SHARED((NP2, HID), jnp.float32),
        ],
    )
    return f(src1d, dst1d, as2, ad2, g)


def _tc_g_body(s2_ref, b2_ref, out_ref):
    s = s2_ref[0] + s2_ref[1]
    bn = s.shape[0]
    re = 1.0 / (s[:, 40:41] + 1e-16)
    ro = 1.0 / (s[:, 104:105] + 1e-16)
    rfull = jnp.concatenate(
        [jnp.broadcast_to(re, (bn, 64)), jnp.broadcast_to(ro, (bn, 64))],
        axis=1)
    out_ref[...] = s * rfull + b2_ref[...]


def _tc_g(s2, b2m2):
    bn = 512
    grid = (NP2 // bn,)
    return pl.pallas_call(
        _tc_g_body,
        grid=grid,
        in_specs=[
            pl.BlockSpec((NCORES, bn, HID), lambda i: (0, i, 0)),
            pl.BlockSpec((1, HID), lambda i: (0, 0)),
        ],
        out_specs=pl.BlockSpec((bn, HID), lambda i: (i, 0)),
        out_shape=jax.ShapeDtypeStruct((NP2, HID), jnp.float32),
    )(s2, b2m2)


def kernel(x, edge_index, W1, a_src1, a_dst1, b1, W2, a_src2, a_dst2, b2):
    loop = jnp.arange(N, dtype=edge_index.dtype)
    pad = jnp.zeros((EP - EREAL,), edge_index.dtype)
    src1d = jnp.concatenate([edge_index[0], loop, pad])
    dst1d = jnp.concatenate([edge_index[1], loop, pad])

    x_p = jnp.concatenate([x, jnp.zeros((N2 - N, D_IN), jnp.float32)], axis=0)
    eye = jnp.eye(H1, dtype=jnp.float32)
    asm = (eye[:, :, None] * a_src1[None, :, :]).reshape(H1, W1OUT)
    adm = (eye[:, :, None] * a_dst1[None, :, :]).reshape(H1, W1OUT)

    h_all, ast, adt = _tc_a(x_p, W1, asm, adm)
    hflat = h_all.reshape(H1 * N2, HID)

    s1, den1 = _sc_c1(src1d, dst1d, ast, adt, hflat)
    s1 = jnp.concatenate(
        [s1, jnp.zeros((H1, N2 - NACC, HID), jnp.float32)], axis=1)

    b1m = b1.reshape(H1, HID)
    W2p = jnp.concatenate(
        [W2, jnp.zeros((W1OUT, HID - NUM_CLASSES), jnp.float32)], axis=1)
    a2sp = jnp.concatenate(
        [a_src2[0], jnp.zeros((HID - NUM_CLASSES,), jnp.float32)]
    ).reshape(HID, 1)
    a2dp = jnp.concatenate(
        [a_dst2[0], jnp.zeros((HID - NUM_CLASSES,), jnp.float32)]
    ).reshape(HID, 1)

    dent = den1.reshape(H1, DND * 128)[:, :N2].T
    g, as2, ad2 = _tc_d(s1, dent, b1m, W2p, a2sp, a2dp)

    s2 = _sc_c2(src1d, dst1d, as2.reshape(N2), ad2.reshape(N2), g)

    op = _tc_g(s2, jnp.concatenate([b2, jnp.zeros((24,), jnp.float32),
                                    b2, jnp.zeros((24,), jnp.float32)]
                                   ).reshape(1, HID))
    out = jnp.stack([op[:, 0:NUM_CLASSES], op[:, 64:64 + NUM_CLASSES]],
                    axis=1).reshape(N2, NUM_CLASSES)
    return out[:N]
